# speedup vs baseline: 1.1250x; 1.1250x over previous
import functools

import jax
import jax.numpy as jnp
from jax.experimental import pallas as pl
from jax.experimental.pallas import tpu as pltpu

_NEG_PAD = -1e30


def _round_up(x, m):
    return (x + m - 1) // m * m


def _bigram_kernel(idx_ref, tgt_ref, tab_bf16_ref, tab_f32_ref,
                   logits_ref, loss_ref, pair_acc,
                   *, inner, bt_total, tile_rows, need_row_mask):
    i = pl.program_id(0)
    j = pl.program_id(1)
    tm, vp = logits_ref.shape

    lane = jax.lax.broadcasted_iota(jnp.int32, (tm, vp), 1)
    oh_idx = (lane == idx_ref[...]).astype(jnp.bfloat16)
    oh_tgt = (lane == tgt_ref[...]).astype(jnp.bfloat16)

    if need_row_mask:
        tile = i * inner + j
        row = tile * tile_rows + jax.lax.broadcasted_iota(
            jnp.int32, (tm, 1), 0)
        valid = row < bt_total
        oh_idx = jnp.where(valid, oh_idx, jnp.bfloat16(0))
        oh_tgt = jnp.where(valid, oh_tgt, jnp.bfloat16(0))

    logits_ref[...] = jnp.dot(oh_idx, tab_bf16_ref[...],
                              preferred_element_type=jnp.float32)

    pair = jax.lax.dot_general(
        oh_idx, oh_tgt, (((0,), (0,)), ((), ())),
        preferred_element_type=jnp.float32)

    @pl.when(j == 0)
    def _():
        pair_acc[...] = pair

    @pl.when(j > 0)
    def _():
        pair_acc[...] += pair

    @pl.when(j == inner - 1)
    def _():
        tab = tab_f32_ref[...]
        m = jnp.max(tab, axis=1, keepdims=True)
        lse = m + jnp.log(jnp.sum(jnp.exp(tab - m), axis=1, keepdims=True))
        contrib = jnp.sum(pair_acc[...] * (lse - tab))
        r = jax.lax.broadcasted_iota(jnp.int32, loss_ref.shape, 1)
        c = jax.lax.broadcasted_iota(jnp.int32, loss_ref.shape, 2)
        loss_ref[...] = jnp.where((r == 0) & (c == 0), contrib, 0.0)


def kernel(idx, table, targets):
    B, T = idx.shape
    V = table.shape[0]
    BT = B * T

    Vp = _round_up(max(V, 128), 128)
    if Vp != V:
        top = jnp.concatenate(
            [table.astype(jnp.float32),
             jnp.full((V, Vp - V), _NEG_PAD, jnp.float32)], axis=1)
        table_p = jnp.concatenate(
            [top, jnp.zeros((Vp - V, Vp), jnp.float32)], axis=0)
    else:
        table_p = table.astype(jnp.float32)
    table_bf16 = table_p.astype(jnp.bfloat16)

    tile_rows = 1024
    tm = tile_rows if BT >= tile_rows else _round_up(max(BT, 8), 8)
    bt_pad = _round_up(BT, tm)
    num_tiles = bt_pad // tm
    ncores = 2 if num_tiles % 2 == 0 else 1
    inner = num_tiles // ncores
    need_row_mask = bt_pad != BT

    idx_col = idx.reshape(BT).astype(jnp.int32)
    tgt_col = targets.reshape(BT).astype(jnp.int32)
    if need_row_mask:
        idx_col = jnp.pad(idx_col, (0, bt_pad - BT))
        tgt_col = jnp.pad(tgt_col, (0, bt_pad - BT))
    idx_col = idx_col[:, None]
    tgt_col = tgt_col[:, None]

    body = functools.partial(
        _bigram_kernel, inner=inner, bt_total=BT, tile_rows=tm,
        need_row_mask=need_row_mask)

    logits_p, loss_parts = pl.pallas_call(
        body,
        out_shape=(
            jax.ShapeDtypeStruct((bt_pad, Vp), jnp.float32),
            jax.ShapeDtypeStruct((ncores, 8, 128), jnp.float32),
        ),
        grid=(ncores, inner),
        in_specs=[
            pl.BlockSpec((tm, 1), lambda i, j: (i * inner + j, 0)),
            pl.BlockSpec((tm, 1), lambda i, j: (i * inner + j, 0)),
            pl.BlockSpec((Vp, Vp), lambda i, j: (0, 0)),
            pl.BlockSpec((Vp, Vp), lambda i, j: (0, 0)),
        ],
        out_specs=(
            pl.BlockSpec((tm, Vp), lambda i, j: (i * inner + j, 0)),
            pl.BlockSpec((1, 8, 128), lambda i, j: (i, 0, 0)),
        ),
        scratch_shapes=[pltpu.VMEM((Vp, Vp), jnp.float32)],
        compiler_params=pltpu.CompilerParams(
            dimension_semantics=("parallel", "arbitrary")),
    )(idx_col, tgt_col, table_bf16, table_p)

    loss = jnp.sum(loss_parts) / jnp.float32(BT)
    if bt_pad != BT or Vp != V:
        logits_out = logits_p[:BT, :V]
    else:
        logits_out = logits_p
    return logits_out, loss


# speedup vs baseline: 16.4603x; 14.6308x over previous
import functools

import jax
import jax.numpy as jnp
from jax.experimental import pallas as pl
from jax.experimental.pallas import tpu as pltpu

_NEG_PAD = -1e30


def _round_up(x, m):
    return (x + m - 1) // m * m


def _loss_epilogue(tab_f32_ref, loss_ref, pair_acc):
    tab = tab_f32_ref[...]
    m = jnp.max(tab, axis=1, keepdims=True)
    lse = m + jnp.log(jnp.sum(jnp.exp(tab - m), axis=1, keepdims=True))
    contrib = jnp.sum(pair_acc[...] * (lse - tab))
    r = jax.lax.broadcasted_iota(jnp.int32, loss_ref.shape, 1)
    c = jax.lax.broadcasted_iota(jnp.int32, loss_ref.shape, 2)
    loss_ref[...] = jnp.where((r == 0) & (c == 0), contrib, 0.0)


def _bigram_rows_kernel(idx_ref, tgt_ref, tab_bf16_ref, tab_f32_ref,
                        logits_ref, loss_ref, pair_acc, *, inner, bm):
    j = pl.program_id(1)
    vp = tab_bf16_ref.shape[0]
    t = idx_ref.shape[1]

    siota = jax.lax.broadcasted_iota(jnp.int32, (vp, t), 0)
    pair_sum = None
    for k in range(bm):
        oh_idx = (siota == idx_ref[k:k + 1, :]).astype(jnp.bfloat16)
        oh_tgt = (siota == tgt_ref[k:k + 1, :]).astype(jnp.bfloat16)
        logits_ref[k] = jax.lax.dot_general(
            oh_idx, tab_bf16_ref[...], (((0,), (0,)), ((), ())),
            preferred_element_type=jnp.float32)
        ck = jax.lax.dot_general(
            oh_idx, oh_tgt, (((1,), (1,)), ((), ())),
            preferred_element_type=jnp.float32)
        pair_sum = ck if pair_sum is None else pair_sum + ck

    @pl.when(j == 0)
    def _():
        pair_acc[...] = pair_sum

    @pl.when(j > 0)
    def _():
        pair_acc[...] += pair_sum

    @pl.when(j == inner - 1)
    def _():
        _loss_epilogue(tab_f32_ref, loss_ref, pair_acc)


def _bigram_cols_kernel(idx_ref, tgt_ref, tab_bf16_ref, tab_f32_ref,
                        logits_ref, loss_ref, pair_acc,
                        *, inner, bt_total, tile_rows, need_row_mask):
    i = pl.program_id(0)
    j = pl.program_id(1)
    tm, vp = logits_ref.shape

    lane = jax.lax.broadcasted_iota(jnp.int32, (tm, vp), 1)
    oh_idx = (lane == idx_ref[...]).astype(jnp.bfloat16)
    oh_tgt = (lane == tgt_ref[...]).astype(jnp.bfloat16)

    if need_row_mask:
        tile = i * inner + j
        row = tile * tile_rows + jax.lax.broadcasted_iota(
            jnp.int32, (tm, 1), 0)
        valid = row < bt_total
        oh_idx = jnp.where(valid, oh_idx, jnp.bfloat16(0))
        oh_tgt = jnp.where(valid, oh_tgt, jnp.bfloat16(0))

    logits_ref[...] = jnp.dot(oh_idx, tab_bf16_ref[...],
                              preferred_element_type=jnp.float32)
    pair = jax.lax.dot_general(
        oh_idx, oh_tgt, (((0,), (0,)), ((), ())),
        preferred_element_type=jnp.float32)

    @pl.when(j == 0)
    def _():
        pair_acc[...] = pair

    @pl.when(j > 0)
    def _():
        pair_acc[...] += pair

    @pl.when(j == inner - 1)
    def _():
        _loss_epilogue(tab_f32_ref, loss_ref, pair_acc)


def _padded_tables(table, V, Vp):
    if Vp != V:
        top = jnp.concatenate(
            [table.astype(jnp.float32),
             jnp.full((V, Vp - V), _NEG_PAD, jnp.float32)], axis=1)
        table_p = jnp.concatenate(
            [top, jnp.zeros((Vp - V, Vp), jnp.float32)], axis=0)
    else:
        table_p = table.astype(jnp.float32)
    return table_p, table_p.astype(jnp.bfloat16)


def kernel(idx, table, targets):
    B, T = idx.shape
    V = table.shape[0]
    BT = B * T
    Vp = _round_up(max(V, 128), 128)
    table_p, table_bf16 = _padded_tables(table, V, Vp)

    bm = 8
    fast = (Vp == 128 and T % 128 == 0 and T >= 128 and B % bm == 0)
    if fast:
        nb = B // bm
        ncores = 2 if nb % 2 == 0 else 1
        inner = nb // ncores

        body = functools.partial(_bigram_rows_kernel, inner=inner, bm=bm)
        logits3, loss_parts = pl.pallas_call(
            body,
            out_shape=(
                jax.ShapeDtypeStruct((B, T, Vp), jnp.float32),
                jax.ShapeDtypeStruct((ncores, 8, 128), jnp.float32),
            ),
            grid=(ncores, inner),
            in_specs=[
                pl.BlockSpec((bm, T), lambda i, j: (i * inner + j, 0)),
                pl.BlockSpec((bm, T), lambda i, j: (i * inner + j, 0)),
                pl.BlockSpec((Vp, Vp), lambda i, j: (0, 0)),
                pl.BlockSpec((Vp, Vp), lambda i, j: (0, 0)),
            ],
            out_specs=(
                pl.BlockSpec((bm, T, Vp), lambda i, j: (i * inner + j, 0, 0)),
                pl.BlockSpec((1, 8, 128), lambda i, j: (i, 0, 0)),
            ),
            scratch_shapes=[pltpu.VMEM((Vp, Vp), jnp.float32)],
            compiler_params=pltpu.CompilerParams(
                dimension_semantics=("parallel", "arbitrary")),
        )(idx.astype(jnp.int32), targets.astype(jnp.int32),
          table_bf16, table_p)

        loss = jnp.sum(loss_parts) / jnp.float32(BT)
        return logits3.reshape(BT, Vp)[:, :V], loss

    tile_rows = 1024
    tm = tile_rows if BT >= tile_rows else _round_up(max(BT, 8), 8)
    bt_pad = _round_up(BT, tm)
    num_tiles = bt_pad // tm
    ncores = 2 if num_tiles % 2 == 0 else 1
    inner = num_tiles // ncores
    need_row_mask = bt_pad != BT

    idx_col = idx.reshape(BT).astype(jnp.int32)
    tgt_col = targets.reshape(BT).astype(jnp.int32)
    if need_row_mask:
        idx_col = jnp.pad(idx_col, (0, bt_pad - BT))
        tgt_col = jnp.pad(tgt_col, (0, bt_pad - BT))
    idx_col = idx_col[:, None]
    tgt_col = tgt_col[:, None]

    body = functools.partial(
        _bigram_cols_kernel, inner=inner, bt_total=BT, tile_rows=tm,
        need_row_mask=need_row_mask)
    logits_p, loss_parts = pl.pallas_call(
        body,
        out_shape=(
            jax.ShapeDtypeStruct((bt_pad, Vp), jnp.float32),
            jax.ShapeDtypeStruct((ncores, 8, 128), jnp.float32),
        ),
        grid=(ncores, inner),
        in_specs=[
            pl.BlockSpec((tm, 1), lambda i, j: (i * inner + j, 0)),
            pl.BlockSpec((tm, 1), lambda i, j: (i * inner + j, 0)),
            pl.BlockSpec((Vp, Vp), lambda i, j: (0, 0)),
            pl.BlockSpec((Vp, Vp), lambda i, j: (0, 0)),
        ],
        out_specs=(
            pl.BlockSpec((tm, Vp), lambda i, j: (i * inner + j, 0)),
            pl.BlockSpec((1, 8, 128), lambda i, j: (i, 0, 0)),
        ),
        scratch_shapes=[pltpu.VMEM((Vp, Vp), jnp.float32)],
        compiler_params=pltpu.CompilerParams(
            dimension_semantics=("parallel", "arbitrary")),
    )(idx_col, tgt_col, table_bf16, table_p)

    loss = jnp.sum(loss_parts) / jnp.float32(BT)
    if bt_pad != BT or Vp != V:
        logits_out = logits_p[:BT, :V]
    else:
        logits_out = logits_p
    return logits_out, loss


# speedup vs baseline: 19.0867x; 1.1596x over previous
import functools

import jax
import jax.numpy as jnp
from jax.experimental import pallas as pl
from jax.experimental.pallas import tpu as pltpu

_NEG_PAD = -1e30


def _round_up(x, m):
    return (x + m - 1) // m * m


def _loss_epilogue(tab_f32_ref, loss_ref, pair_acc):
    tab = tab_f32_ref[...]
    m = jnp.max(tab, axis=1, keepdims=True)
    lse = m + jnp.log(jnp.sum(jnp.exp(tab - m), axis=1, keepdims=True))
    contrib = jnp.sum(pair_acc[...] * (lse - tab))
    r = jax.lax.broadcasted_iota(jnp.int32, loss_ref.shape, 1)
    c = jax.lax.broadcasted_iota(jnp.int32, loss_ref.shape, 2)
    loss_ref[...] = jnp.where((r == 0) & (c == 0), contrib, 0.0)


def _bigram_rows_kernel(idx_ref, tgt_ref, tab_bf16_ref, tab_f32_ref,
                        logits_ref, loss_ref, pair_acc, *, inner, bm):
    j = pl.program_id(1)
    vp = tab_bf16_ref.shape[0]
    t = idx_ref.shape[1]

    siota = jax.lax.broadcasted_iota(jnp.int32, (vp, t), 0)
    pair_sum = None
    for k in range(bm):
        oh_idx = (siota == idx_ref[k:k + 1, :]).astype(jnp.bfloat16)
        oh_tgt = (siota == tgt_ref[k:k + 1, :]).astype(jnp.bfloat16)
        logits_ref[k] = jax.lax.dot_general(
            oh_idx, tab_bf16_ref[...], (((0,), (0,)), ((), ())),
            preferred_element_type=jnp.float32)
        ck = jax.lax.dot_general(
            oh_idx, oh_tgt, (((1,), (1,)), ((), ())),
            preferred_element_type=jnp.float32)
        pair_sum = ck if pair_sum is None else pair_sum + ck

    @pl.when(j == 0)
    def _():
        pair_acc[...] = pair_sum

    @pl.when(j > 0)
    def _():
        pair_acc[...] += pair_sum

    @pl.when(j == inner - 1)
    def _():
        _loss_epilogue(tab_f32_ref, loss_ref, pair_acc)


def _bigram_cols_kernel(idx_ref, tgt_ref, tab_bf16_ref, tab_f32_ref,
                        logits_ref, loss_ref, pair_acc,
                        *, inner, bt_total, tile_rows, need_row_mask):
    i = pl.program_id(0)
    j = pl.program_id(1)
    tm, vp = logits_ref.shape

    lane = jax.lax.broadcasted_iota(jnp.int32, (tm, vp), 1)
    oh_idx = (lane == idx_ref[...]).astype(jnp.bfloat16)
    oh_tgt = (lane == tgt_ref[...]).astype(jnp.bfloat16)

    if need_row_mask:
        tile = i * inner + j
        row = tile * tile_rows + jax.lax.broadcasted_iota(
            jnp.int32, (tm, 1), 0)
        valid = row < bt_total
        oh_idx = jnp.where(valid, oh_idx, jnp.bfloat16(0))
        oh_tgt = jnp.where(valid, oh_tgt, jnp.bfloat16(0))

    logits_ref[...] = jnp.dot(oh_idx, tab_bf16_ref[...],
                              preferred_element_type=jnp.float32)
    pair = jax.lax.dot_general(
        oh_idx, oh_tgt, (((0,), (0,)), ((), ())),
        preferred_element_type=jnp.float32)

    @pl.when(j == 0)
    def _():
        pair_acc[...] = pair

    @pl.when(j > 0)
    def _():
        pair_acc[...] += pair

    @pl.when(j == inner - 1)
    def _():
        _loss_epilogue(tab_f32_ref, loss_ref, pair_acc)


def _padded_tables(table, V, Vp):
    if Vp != V:
        top = jnp.concatenate(
            [table.astype(jnp.float32),
             jnp.full((V, Vp - V), _NEG_PAD, jnp.float32)], axis=1)
        table_p = jnp.concatenate(
            [top, jnp.zeros((Vp - V, Vp), jnp.float32)], axis=0)
    else:
        table_p = table.astype(jnp.float32)
    return table_p, table_p.astype(jnp.bfloat16)


def kernel(idx, table, targets):
    B, T = idx.shape
    V = table.shape[0]
    BT = B * T
    Vp = _round_up(max(V, 128), 128)
    table_p, table_bf16 = _padded_tables(table, V, Vp)

    bm = 16 if B % 16 == 0 else 8
    fast = (Vp == 128 and T % 128 == 0 and T >= 128 and B % bm == 0)
    if fast:
        nb = B // bm
        ncores = 2 if nb % 2 == 0 else 1
        inner = nb // ncores

        body = functools.partial(_bigram_rows_kernel, inner=inner, bm=bm)
        logits3, loss_parts = pl.pallas_call(
            body,
            out_shape=(
                jax.ShapeDtypeStruct((B, T, Vp), jnp.float32),
                jax.ShapeDtypeStruct((ncores, 8, 128), jnp.float32),
            ),
            grid=(ncores, inner),
            in_specs=[
                pl.BlockSpec((bm, T), lambda i, j: (i * inner + j, 0)),
                pl.BlockSpec((bm, T), lambda i, j: (i * inner + j, 0)),
                pl.BlockSpec((Vp, Vp), lambda i, j: (0, 0)),
                pl.BlockSpec((Vp, Vp), lambda i, j: (0, 0)),
            ],
            out_specs=(
                pl.BlockSpec((bm, T, Vp), lambda i, j: (i * inner + j, 0, 0)),
                pl.BlockSpec((1, 8, 128), lambda i, j: (i, 0, 0)),
            ),
            scratch_shapes=[pltpu.VMEM((Vp, Vp), jnp.float32)],
            compiler_params=pltpu.CompilerParams(
                dimension_semantics=("parallel", "arbitrary")),
        )(idx.astype(jnp.int32), targets.astype(jnp.int32),
          table_bf16, table_p)

        loss = jnp.sum(loss_parts) / jnp.float32(BT)
        return logits3.reshape(BT, Vp)[:, :V], loss

    tile_rows = 1024
    tm = tile_rows if BT >= tile_rows else _round_up(max(BT, 8), 8)
    bt_pad = _round_up(BT, tm)
    num_tiles = bt_pad // tm
    ncores = 2 if num_tiles % 2 == 0 else 1
    inner = num_tiles // ncores
    need_row_mask = bt_pad != BT

    idx_col = idx.reshape(BT).astype(jnp.int32)
    tgt_col = targets.reshape(BT).astype(jnp.int32)
    if need_row_mask:
        idx_col = jnp.pad(idx_col, (0, bt_pad - BT))
        tgt_col = jnp.pad(tgt_col, (0, bt_pad - BT))
    idx_col = idx_col[:, None]
    tgt_col = tgt_col[:, None]

    body = functools.partial(
        _bigram_cols_kernel, inner=inner, bt_total=BT, tile_rows=tm,
        need_row_mask=need_row_mask)
    logits_p, loss_parts = pl.pallas_call(
        body,
        out_shape=(
            jax.ShapeDtypeStruct((bt_pad, Vp), jnp.float32),
            jax.ShapeDtypeStruct((ncores, 8, 128), jnp.float32),
        ),
        grid=(ncores, inner),
        in_specs=[
            pl.BlockSpec((tm, 1), lambda i, j: (i * inner + j, 0)),
            pl.BlockSpec((tm, 1), lambda i, j: (i * inner + j, 0)),
            pl.BlockSpec((Vp, Vp), lambda i, j: (0, 0)),
            pl.BlockSpec((Vp, Vp), lambda i, j: (0, 0)),
        ],
        out_specs=(
            pl.BlockSpec((tm, Vp), lambda i, j: (i * inner + j, 0)),
            pl.BlockSpec((1, 8, 128), lambda i, j: (i, 0, 0)),
        ),
        scratch_shapes=[pltpu.VMEM((Vp, Vp), jnp.float32)],
        compiler_params=pltpu.CompilerParams(
            dimension_semantics=("parallel", "arbitrary")),
    )(idx_col, tgt_col, table_bf16, table_p)

    loss = jnp.sum(loss_parts) / jnp.float32(BT)
    if bt_pad != BT or Vp != V:
        logits_out = logits_p[:BT, :V]
    else:
        logits_out = logits_p
    return logits_out, loss


# speedup vs baseline: 20.3553x; 1.0665x over previous
import functools

import jax
import jax.numpy as jnp
from jax.experimental import pallas as pl
from jax.experimental.pallas import tpu as pltpu

_NEG_PAD = -1e30


def _round_up(x, m):
    return (x + m - 1) // m * m


def _loss_epilogue(tab_f32_ref, loss_ref, pair_acc):
    tab = tab_f32_ref[...]
    m = jnp.max(tab, axis=1, keepdims=True)
    lse = m + jnp.log(jnp.sum(jnp.exp(tab - m), axis=1, keepdims=True))
    contrib = jnp.sum(pair_acc[...] * (lse - tab))
    r = jax.lax.broadcasted_iota(jnp.int32, loss_ref.shape, 1)
    c = jax.lax.broadcasted_iota(jnp.int32, loss_ref.shape, 2)
    loss_ref[...] = jnp.where((r == 0) & (c == 0), contrib, 0.0)


def _bigram_rows_kernel(idx_ref, tgt_ref, tab_bf16_ref, tab_f32_ref,
                        logits_ref, loss_ref, pair_acc, *, inner, bm):
    j = pl.program_id(1)
    vp = tab_bf16_ref.shape[0]
    t = idx_ref.shape[1]

    siota = jax.lax.broadcasted_iota(jnp.int32, (vp, t), 0)
    pair_sum = None
    for k in range(bm):
        oh_idx = (siota == idx_ref[k:k + 1, :]).astype(jnp.bfloat16)
        oh_tgt = (siota == tgt_ref[k:k + 1, :]).astype(jnp.bfloat16)
        logits_ref[k] = jax.lax.dot_general(
            oh_idx, tab_bf16_ref[...], (((0,), (0,)), ((), ())),
            preferred_element_type=jnp.float32)
        ck = jax.lax.dot_general(
            oh_idx, oh_tgt, (((1,), (1,)), ((), ())),
            preferred_element_type=jnp.float32)
        pair_sum = ck if pair_sum is None else pair_sum + ck

    @pl.when(j == 0)
    def _():
        pair_acc[...] = pair_sum

    @pl.when(j > 0)
    def _():
        pair_acc[...] += pair_sum

    @pl.when(j == inner - 1)
    def _():
        _loss_epilogue(tab_f32_ref, loss_ref, pair_acc)


def _bigram_cols_kernel(idx_ref, tgt_ref, tab_bf16_ref, tab_f32_ref,
                        logits_ref, loss_ref, pair_acc,
                        *, inner, bt_total, tile_rows, need_row_mask):
    i = pl.program_id(0)
    j = pl.program_id(1)
    tm, vp = logits_ref.shape

    lane = jax.lax.broadcasted_iota(jnp.int32, (tm, vp), 1)
    oh_idx = (lane == idx_ref[...]).astype(jnp.bfloat16)
    oh_tgt = (lane == tgt_ref[...]).astype(jnp.bfloat16)

    if need_row_mask:
        tile = i * inner + j
        row = tile * tile_rows + jax.lax.broadcasted_iota(
            jnp.int32, (tm, 1), 0)
        valid = row < bt_total
        oh_idx = jnp.where(valid, oh_idx, jnp.bfloat16(0))
        oh_tgt = jnp.where(valid, oh_tgt, jnp.bfloat16(0))

    logits_ref[...] = jnp.dot(oh_idx, tab_bf16_ref[...],
                              preferred_element_type=jnp.float32)
    pair = jax.lax.dot_general(
        oh_idx, oh_tgt, (((0,), (0,)), ((), ())),
        preferred_element_type=jnp.float32)

    @pl.when(j == 0)
    def _():
        pair_acc[...] = pair

    @pl.when(j > 0)
    def _():
        pair_acc[...] += pair

    @pl.when(j == inner - 1)
    def _():
        _loss_epilogue(tab_f32_ref, loss_ref, pair_acc)


def _padded_tables(table, V, Vp):
    if Vp != V:
        top = jnp.concatenate(
            [table.astype(jnp.float32),
             jnp.full((V, Vp - V), _NEG_PAD, jnp.float32)], axis=1)
        table_p = jnp.concatenate(
            [top, jnp.zeros((Vp - V, Vp), jnp.float32)], axis=0)
    else:
        table_p = table.astype(jnp.float32)
    return table_p, table_p.astype(jnp.bfloat16)


def kernel(idx, table, targets):
    B, T = idx.shape
    V = table.shape[0]
    BT = B * T
    Vp = _round_up(max(V, 128), 128)
    table_p, table_bf16 = _padded_tables(table, V, Vp)

    bm = 32 if B % 32 == 0 else (16 if B % 16 == 0 else 8)
    fast = (Vp == 128 and T % 128 == 0 and T >= 128 and B % bm == 0)
    if fast:
        nb = B // bm
        ncores = 2 if nb % 2 == 0 else 1
        inner = nb // ncores

        body = functools.partial(_bigram_rows_kernel, inner=inner, bm=bm)
        logits3, loss_parts = pl.pallas_call(
            body,
            out_shape=(
                jax.ShapeDtypeStruct((B, T, Vp), jnp.float32),
                jax.ShapeDtypeStruct((ncores, 8, 128), jnp.float32),
            ),
            grid=(ncores, inner),
            in_specs=[
                pl.BlockSpec((bm, T), lambda i, j: (i * inner + j, 0)),
                pl.BlockSpec((bm, T), lambda i, j: (i * inner + j, 0)),
                pl.BlockSpec((Vp, Vp), lambda i, j: (0, 0)),
                pl.BlockSpec((Vp, Vp), lambda i, j: (0, 0)),
            ],
            out_specs=(
                pl.BlockSpec((bm, T, Vp), lambda i, j: (i * inner + j, 0, 0)),
                pl.BlockSpec((1, 8, 128), lambda i, j: (i, 0, 0)),
            ),
            scratch_shapes=[pltpu.VMEM((Vp, Vp), jnp.float32)],
            compiler_params=pltpu.CompilerParams(
                dimension_semantics=("parallel", "arbitrary")),
        )(idx.astype(jnp.int32), targets.astype(jnp.int32),
          table_bf16, table_p)

        loss = jnp.sum(loss_parts) / jnp.float32(BT)
        return logits3.reshape(BT, Vp)[:, :V], loss

    tile_rows = 1024
    tm = tile_rows if BT >= tile_rows else _round_up(max(BT, 8), 8)
    bt_pad = _round_up(BT, tm)
    num_tiles = bt_pad // tm
    ncores = 2 if num_tiles % 2 == 0 else 1
    inner = num_tiles // ncores
    need_row_mask = bt_pad != BT

    idx_col = idx.reshape(BT).astype(jnp.int32)
    tgt_col = targets.reshape(BT).astype(jnp.int32)
    if need_row_mask:
        idx_col = jnp.pad(idx_col, (0, bt_pad - BT))
        tgt_col = jnp.pad(tgt_col, (0, bt_pad - BT))
    idx_col = idx_col[:, None]
    tgt_col = tgt_col[:, None]

    body = functools.partial(
        _bigram_cols_kernel, inner=inner, bt_total=BT, tile_rows=tm,
        need_row_mask=need_row_mask)
    logits_p, loss_parts = pl.pallas_call(
        body,
        out_shape=(
            jax.ShapeDtypeStruct((bt_pad, Vp), jnp.float32),
            jax.ShapeDtypeStruct((ncores, 8, 128), jnp.float32),
        ),
        grid=(ncores, inner),
        in_specs=[
            pl.BlockSpec((tm, 1), lambda i, j: (i * inner + j, 0)),
            pl.BlockSpec((tm, 1), lambda i, j: (i * inner + j, 0)),
            pl.BlockSpec((Vp, Vp), lambda i, j: (0, 0)),
            pl.BlockSpec((Vp, Vp), lambda i, j: (0, 0)),
        ],
        out_specs=(
            pl.BlockSpec((tm, Vp), lambda i, j: (i * inner + j, 0)),
            pl.BlockSpec((1, 8, 128), lambda i, j: (i, 0, 0)),
        ),
        scratch_shapes=[pltpu.VMEM((Vp, Vp), jnp.float32)],
        compiler_params=pltpu.CompilerParams(
            dimension_semantics=("parallel", "arbitrary")),
    )(idx_col, tgt_col, table_bf16, table_p)

    loss = jnp.sum(loss_parts) / jnp.float32(BT)
    if bt_pad != BT or Vp != V:
        logits_out = logits_p[:BT, :V]
    else:
        logits_out = logits_p
    return logits_out, loss
